# Initial kernel scaffold; baseline (speedup 1.0000x reference)
import jax
import jax.numpy as jnp
from jax.experimental import pallas as pl


def kernel(s_cat, s_cont, k_cat, k_cont, o_cat, o_cont, target, s_tab0, s_tab1, k_tab0, k_tab1, k_tab2, o_tab0, s_cont_w, s_cont_b, k_cont_w, k_cont_b, o_cont_w, o_cont_b, tgt_w, tgt_b):
    raise NotImplementedError("write your pallas kernel here")


# speedup vs baseline: 1.5825x; 1.5825x over previous
import functools

import jax
import jax.numpy as jnp
from jax import lax
from jax.experimental import pallas as pl
from jax.experimental.pallas import tpu as pltpu
from jax.experimental.pallas import tpu_sc as plsc

_B, _T, _H = 1024, 200, 128
_N = _B * _T
_NW = 32
_NPW = _N // _NW
_CH = 128
_NCH = _NPW // _CH
_NBUF = 5


def _sc_gather_body(kidx0, kidx1, kidx2, oidx, sidx0, sidx1,
                    k_tab0, k_tab1, k_tab2, o_tab0, s_tab0, s_tab1,
                    r_k0, r_k1, r_k2, r_o0, r_s0, r_s1,
                    idx_v, rows, gsem, ssem):
    wid = lax.axis_index("s") * 2 + lax.axis_index("c")

    @pl.when(wid < 8)
    def _():
        pltpu.sync_copy(sidx0.at[wid], idx_v.at[pl.ds(0, 1)])
        pltpu.async_copy(s_tab0.at[idx_v.at[0]], rows.at[0], gsem.at[0]).wait()
        pltpu.sync_copy(rows.at[0], r_s0.at[pl.ds(wid * _CH, _CH)])

    @pl.when((wid >= 8) & (wid < 16))
    def _():
        w = wid - 8
        pltpu.sync_copy(sidx1.at[w], idx_v.at[pl.ds(0, 1)])
        pltpu.async_copy(s_tab1.at[idx_v.at[0]], rows.at[0], gsem.at[0]).wait()
        pltpu.sync_copy(rows.at[0], r_s1.at[pl.ds(w * _CH, _CH)])

    base = wid * _NPW

    for idx_hbm, tab, out in ((kidx0, k_tab0, r_k0), (kidx1, k_tab1, r_k1),
                              (kidx2, k_tab2, r_k2), (oidx, o_tab0, r_o0)):
        pltpu.sync_copy(idx_hbm.at[wid], idx_v)
        for b in range(_NBUF):
            pltpu.async_copy(tab.at[idx_v.at[b]], rows.at[b], gsem.at[b])

        @pl.loop(0, _NCH - _NBUF, step=_NBUF)
        def _(g):
            for b in range(_NBUF):
                c = g + b
                pltpu.make_async_copy(tab.at[idx_v.at[c]], rows.at[b],
                                      gsem.at[b]).wait()
                pltpu.async_copy(rows.at[b],
                                 out.at[pl.ds(base + c * _CH, _CH)], ssem.at[b])
            for b in range(_NBUF):
                c = g + b
                pltpu.make_async_copy(rows.at[b],
                                      out.at[pl.ds(base + c * _CH, _CH)],
                                      ssem.at[b]).wait()
                pltpu.async_copy(tab.at[idx_v.at[c + _NBUF]], rows.at[b],
                                 gsem.at[b])

        for b in range(_NBUF):
            c = _NCH - _NBUF + b
            pltpu.make_async_copy(tab.at[idx_v.at[c]], rows.at[b],
                                  gsem.at[b]).wait()
            pltpu.async_copy(rows.at[b],
                             out.at[pl.ds(base + c * _CH, _CH)], ssem.at[b])
        for b in range(_NBUF):
            c = _NCH - _NBUF + b
            pltpu.make_async_copy(rows.at[b],
                                  out.at[pl.ds(base + c * _CH, _CH)],
                                  ssem.at[b]).wait()


_sc_gather = functools.partial(
    pl.kernel,
    out_type=[
        jax.ShapeDtypeStruct((_N, _H), jnp.float32),
        jax.ShapeDtypeStruct((_N, _H), jnp.float32),
        jax.ShapeDtypeStruct((_N, _H), jnp.float32),
        jax.ShapeDtypeStruct((_N, _H), jnp.float32),
        jax.ShapeDtypeStruct((_B, _H), jnp.float32),
        jax.ShapeDtypeStruct((_B, _H), jnp.float32),
    ],
    mesh=plsc.VectorSubcoreMesh(core_axis_name="c", subcore_axis_name="s"),
    scratch_types=[
        pltpu.VMEM((_NCH, _CH), jnp.int32),
        pltpu.VMEM((_NBUF, _CH, _H), jnp.float32),
        pltpu.SemaphoreType.DMA((_NBUF,)),
        pltpu.SemaphoreType.DMA((_NBUF,)),
    ],
)(_sc_gather_body)


_R = 1024


def _tc_assemble_body(k0, k1, k2, o0, kc, oc, tg, kw, kb, ow, ob, tw, tb,
                      out_k, out_o, out_t):
    for j, r in enumerate((k0, k1, k2)):
        out_k[:, j, :] = r[:, :]
    kcv = kc[:, :]
    for f in range(8):
        out_k[:, 3 + f, :] = kcv[:, f:f + 1] * kw[f:f + 1, :] + kb[f:f + 1, :]
    out_o[:, 0, :] = o0[:, :]
    ocv = oc[:, :]
    for f in range(8):
        out_o[:, 1 + f, :] = ocv[:, f:f + 1] * ow[f:f + 1, :] + ob[f:f + 1, :]
    out_t[:, 0, :] = tg[:, 0:1] * tw[0:1, :] + tb[0:1, :]


def _tc_assemble(r_k0, r_k1, r_k2, r_o0, kc, oc, tg, kw, kb, ow, ob, tw, tb):
    row_spec = pl.BlockSpec((_R, _H), lambda i: (i, 0))
    return pl.pallas_call(
        _tc_assemble_body,
        grid=(_N // _R,),
        in_specs=[
            row_spec, row_spec, row_spec, row_spec,
            pl.BlockSpec((_R, 8), lambda i: (i, 0)),
            pl.BlockSpec((_R, 8), lambda i: (i, 0)),
            pl.BlockSpec((_R, 1), lambda i: (i, 0)),
            pl.BlockSpec((8, _H), lambda i: (0, 0)),
            pl.BlockSpec((8, _H), lambda i: (0, 0)),
            pl.BlockSpec((8, _H), lambda i: (0, 0)),
            pl.BlockSpec((8, _H), lambda i: (0, 0)),
            pl.BlockSpec((1, _H), lambda i: (0, 0)),
            pl.BlockSpec((1, _H), lambda i: (0, 0)),
        ],
        out_specs=[
            pl.BlockSpec((_R, 11, _H), lambda i: (i, 0, 0)),
            pl.BlockSpec((_R, 9, _H), lambda i: (i, 0, 0)),
            pl.BlockSpec((_R, 1, _H), lambda i: (i, 0, 0)),
        ],
        out_shape=[
            jax.ShapeDtypeStruct((_N, 11, _H), jnp.float32),
            jax.ShapeDtypeStruct((_N, 9, _H), jnp.float32),
            jax.ShapeDtypeStruct((_N, 1, _H), jnp.float32),
        ],
    )(r_k0, r_k1, r_k2, r_o0, kc, oc, tg, kw, kb, ow, ob, tw, tb)


def _tc_static_body(s0, s1, sc, sw, sb, out):
    out[:, 0, :] = s0[:, :]
    out[:, 1, :] = s1[:, :]
    for f in range(4):
        out[:, 2 + f, :] = sc[:, f:f + 1] * sw[f:f + 1, :] + sb[f:f + 1, :]


def _tc_static(r_s0, r_s1, sc, sw, sb):
    return pl.pallas_call(
        _tc_static_body,
        out_shape=jax.ShapeDtypeStruct((_B, 6, _H), jnp.float32),
    )(r_s0, r_s1, sc, sw, sb)


def kernel(s_cat, s_cont, k_cat, k_cont, o_cat, o_cont, target,
           s_tab0, s_tab1, k_tab0, k_tab1, k_tab2, o_tab0,
           s_cont_w, s_cont_b, k_cont_w, k_cont_b, o_cont_w, o_cont_b,
           tgt_w, tgt_b):
    i32 = jnp.int32
    kidx0 = k_cat[:, :, 0].reshape(_NW, _NCH, _CH).astype(i32)
    kidx1 = k_cat[:, :, 1].reshape(_NW, _NCH, _CH).astype(i32)
    kidx2 = k_cat[:, :, 2].reshape(_NW, _NCH, _CH).astype(i32)
    oidx = o_cat[:, :, 0].reshape(_NW, _NCH, _CH).astype(i32)
    sidx0 = s_cat[:, 0, 0].reshape(_B // _CH, 1, _CH).astype(i32)
    sidx1 = s_cat[:, 0, 1].reshape(_B // _CH, 1, _CH).astype(i32)

    r_k0, r_k1, r_k2, r_o0, r_s0, r_s1 = _sc_gather(
        kidx0, kidx1, kidx2, oidx, sidx0, sidx1,
        k_tab0, k_tab1, k_tab2, o_tab0, s_tab0, s_tab1)

    out_k, out_o, out_t = _tc_assemble(
        r_k0, r_k1, r_k2, r_o0,
        k_cont.reshape(_N, 8), o_cont.reshape(_N, 8), target.reshape(_N, 1),
        k_cont_w, k_cont_b, o_cont_w, o_cont_b, tgt_w, tgt_b)

    s_inp = _tc_static(r_s0, r_s1, s_cont[:, 0, :], s_cont_w, s_cont_b)

    return (s_inp,
            out_k.reshape(_B, _T, 11, _H),
            out_o.reshape(_B, _T, 9, _H),
            out_t.reshape(_B, _T, 1, _H))
